# baseline (speedup 1.0000x reference)
import jax
import jax.numpy as jnp
from jax.experimental import pallas as pl
from jax.experimental.pallas import tpu as pltpu

B = 64
N = 8732
C = 81
NEG_POS_RATIO = 3
BB = 8
NB = 2944
NCHUNK = 3
NT = NCHUNK * NB
NGRP = B // BB


def _fused_kernel(lg_ref, lab_ref, pb_ref, gb_ref,
                  bl_ref, cl_ref, tot_ref,
                  key_s, ce_s, npos_s, npt_s, bsum_s, cpos_s, cneg_s):
    i = pl.program_id(0)
    j = pl.program_id(1)

    x = lg_ref[...]
    m = jnp.max(x, axis=0)
    e = jnp.exp(x - m[None])
    lse = m + jnp.log(jnp.sum(e, axis=0))
    bg = lse - x[0]

    lab = lab_ref[...]
    cls_iota = jax.lax.broadcasted_iota(jnp.int32, (C, BB, NB), 0)
    xl = jnp.sum(jnp.where(cls_iota == lab[None], x, 0.0), axis=0)
    ce = lse - xl

    lane = jax.lax.broadcasted_iota(jnp.int32, (BB, NB), 1)
    valid = (j * NB + lane) < N
    pos = lab > 0
    vpos = valid & pos

    key_s[:, pl.ds(j * NB, NB)] = jnp.where(vpos | ~valid, -1.0, bg)
    ce_s[:, pl.ds(j * NB, NB)] = ce

    d = pb_ref[...] - gb_ref[...]
    ad = jnp.abs(d)
    sl1 = jnp.where(ad < 1.0, 0.5 * d * d, ad - 0.5)
    slr = jnp.sum(sl1, axis=1)

    np_b = jnp.sum(jnp.where(vpos, 1, 0), axis=1, keepdims=True)
    cp_b = jnp.sum(jnp.where(vpos, ce, 0.0)).reshape(1, 1)
    bs_b = jnp.sum(jnp.where(vpos, slr, 0.0)).reshape(1, 1)

    @pl.when(j == 0)
    def _():
        npos_s[...] = jnp.zeros((BB, 1), jnp.int32)

    @pl.when((i == 0) & (j == 0))
    def _():
        npt_s[...] = jnp.zeros((1, 1), jnp.int32)
        bsum_s[...] = jnp.zeros((1, 1), jnp.float32)
        cpos_s[...] = jnp.zeros((1, 1), jnp.float32)
        cneg_s[...] = jnp.zeros((1, 1), jnp.float32)

    npos_s[...] += np_b
    bsum_s[...] += bs_b
    cpos_s[...] += cp_b

    @pl.when(j == NCHUNK - 1)
    def _():
        key = key_s[...]
        ikey = jax.lax.bitcast_convert_type(key, jnp.int32)
        npos_row = npos_s[...]
        k = npos_row * NEG_POS_RATIO

        lo0 = jnp.full((BB, 1), jnp.int32(-1082130432))
        hi0 = jnp.max(ikey, axis=1, keepdims=True)

        def body_v(_, carry):
            lo, hi = carry
            mid = lo + ((hi - lo) >> 1)
            cnt = jnp.sum((ikey > mid).astype(jnp.int32), axis=1,
                          keepdims=True)
            active = lo < hi
            lo = jnp.where(active & (cnt >= k), mid + 1, lo)
            hi = jnp.where(active & (cnt < k), mid, hi)
            return lo, hi

        v, _ = jax.lax.fori_loop(0, 32, body_v, (lo0, hi0))

        cnt_gt = jnp.sum((ikey > v).astype(jnp.int32), axis=1, keepdims=True)
        mm = k - cnt_gt
        eq = ikey == v
        idx = jax.lax.broadcasted_iota(jnp.int32, (BB, NT), 1)

        def body_t(_, carry):
            lo, hi = carry
            mid = lo + ((hi - lo) >> 1)
            cnt = jnp.sum((eq & (idx < mid)).astype(jnp.int32), axis=1,
                          keepdims=True)
            active = lo < hi
            lo = jnp.where(active & (cnt < mm), mid + 1, lo)
            hi = jnp.where(active & (cnt >= mm), mid, hi)
            return lo, hi

        t, _ = jax.lax.fori_loop(0, 14, body_t,
                                 (jnp.zeros((BB, 1), jnp.int32),
                                  jnp.full((BB, 1), jnp.int32(NT))))

        sel_neg = ((ikey > v) | (eq & (idx < t))) & (ikey >= 0)
        cneg_s[...] += jnp.sum(jnp.where(sel_neg, ce_s[...], 0.0)
                               ).reshape(1, 1)
        npt_s[...] += jnp.sum(npos_row).reshape(1, 1)

    @pl.when((i == NGRP - 1) & (j == NCHUNK - 1))
    def _():
        npf = npt_s[0, 0].astype(jnp.float32)
        boxes_loss = bsum_s[...] / npf
        cls_loss = (cpos_s[...] + cneg_s[...]) / npf
        bl_ref[...] = boxes_loss
        cl_ref[...] = cls_loss
        tot_ref[...] = boxes_loss + cls_loss


def kernel(pred_boxes, pred_logits, gt_boxes, gt_labels):
    labels = gt_labels.astype(jnp.int32)
    lgT = jnp.transpose(pred_logits, (2, 0, 1))
    pbT = jnp.transpose(pred_boxes, (0, 2, 1))
    gbT = jnp.transpose(gt_boxes, (0, 2, 1))

    bl, cl, tot = pl.pallas_call(
        _fused_kernel,
        grid=(NGRP, NCHUNK),
        in_specs=[
            pl.BlockSpec((C, BB, NB), lambda i, j: (0, i, j)),
            pl.BlockSpec((BB, NB), lambda i, j: (i, j)),
            pl.BlockSpec((BB, 4, NB), lambda i, j: (i, 0, j)),
            pl.BlockSpec((BB, 4, NB), lambda i, j: (i, 0, j)),
        ],
        out_specs=[
            pl.BlockSpec((1, 1), lambda i, j: (0, 0)),
            pl.BlockSpec((1, 1), lambda i, j: (0, 0)),
            pl.BlockSpec((1, 1), lambda i, j: (0, 0)),
        ],
        out_shape=[
            jax.ShapeDtypeStruct((1, 1), jnp.float32),
            jax.ShapeDtypeStruct((1, 1), jnp.float32),
            jax.ShapeDtypeStruct((1, 1), jnp.float32),
        ],
        scratch_shapes=[
            pltpu.VMEM((BB, NT), jnp.float32),
            pltpu.VMEM((BB, NT), jnp.float32),
            pltpu.VMEM((BB, 1), jnp.int32),
            pltpu.VMEM((1, 1), jnp.int32),
            pltpu.VMEM((1, 1), jnp.float32),
            pltpu.VMEM((1, 1), jnp.float32),
            pltpu.VMEM((1, 1), jnp.float32),
        ],
    )(lgT, labels, pbT, gbT)

    return (bl[0, 0], cl[0, 0], tot[0, 0])


# speedup vs baseline: 1.4107x; 1.4107x over previous
import jax
import jax.numpy as jnp
from jax.experimental import pallas as pl

B = 64
N = 8732
C = 81
NEG_POS_RATIO = 3
BB = 8
NB = 2944
NCHUNK = 3


def _pass1_kernel(lg_ref, lab_ref, pb_ref, gb_ref,
                  key_ref, npos_ref, bsum_ref, cpos_ref):
    i = pl.program_id(0)
    j = pl.program_id(1)

    x = lg_ref[...]
    m = jnp.max(x, axis=0)
    e = jnp.exp(x - m[None])
    lse = m + jnp.log(jnp.sum(e, axis=0))
    bg = lse - x[0]

    lab = lab_ref[...]
    cls_iota = jax.lax.broadcasted_iota(jnp.int32, (C, BB, NB), 0)
    xl = jnp.sum(jnp.where(cls_iota == lab[None], x, 0.0), axis=0)
    ce = lse - xl

    lane = jax.lax.broadcasted_iota(jnp.int32, (BB, NB), 1)
    valid = (j * NB + lane) < N
    pos = lab > 0
    vpos = valid & pos

    key_ref[...] = jnp.where(pos, -1.0, bg)

    d = pb_ref[...] - gb_ref[...]
    ad = jnp.abs(d)
    sl1 = jnp.where(ad < 1.0, 0.5 * d * d, ad - 0.5)
    slr = jnp.sum(sl1, axis=1)

    np_b = jnp.sum(jnp.where(vpos, 1, 0), axis=1, keepdims=True)
    cp_b = jnp.sum(jnp.where(vpos, ce, 0.0)).reshape(1, 1)
    bs_b = jnp.sum(jnp.where(vpos, slr, 0.0)).reshape(1, 1)

    @pl.when(j == 0)
    def _():
        npos_ref[...] = jnp.zeros((BB, 1), jnp.int32)

    @pl.when((i == 0) & (j == 0))
    def _():
        bsum_ref[...] = jnp.zeros((1, 1), jnp.float32)
        cpos_ref[...] = jnp.zeros((1, 1), jnp.float32)

    npos_ref[...] += np_b
    bsum_ref[...] += bs_b
    cpos_ref[...] += cp_b


def _pass2_kernel(key_ref, npos_ref, bsum_ref, cpos_ref,
                  bl_ref, cl_ref, tot_ref):
    key = key_ref[...]
    ikey = jax.lax.bitcast_convert_type(key, jnp.int32)
    npos_row = npos_ref[...]
    k = npos_row * NEG_POS_RATIO

    lo0 = jnp.full((B, 1), jnp.int32(-1082130432))
    hi0 = jnp.max(ikey, axis=1, keepdims=True)

    def body_v(_, carry):
        lo, hi = carry
        mid = lo + ((hi - lo) >> 1)
        cnt = jnp.sum((ikey > mid).astype(jnp.int32), axis=1, keepdims=True)
        active = lo < hi
        lo = jnp.where(active & (cnt >= k), mid + 1, lo)
        hi = jnp.where(active & (cnt < k), mid, hi)
        return lo, hi

    v, _ = jax.lax.fori_loop(0, 32, body_v, (lo0, hi0))

    cnt_gt = jnp.sum((ikey > v).astype(jnp.int32), axis=1, keepdims=True)
    m = k - cnt_gt
    eq = ikey == v

    def body_t(_, carry):
        lo, hi = carry
        mid = lo + ((hi - lo) >> 1)
        idx = jax.lax.broadcasted_iota(jnp.int32, (B, N), 1)
        cnt = jnp.sum((eq & (idx < mid)).astype(jnp.int32), axis=1,
                      keepdims=True)
        active = lo < hi
        lo = jnp.where(active & (cnt < m), mid + 1, lo)
        hi = jnp.where(active & (cnt >= m), mid, hi)
        return lo, hi

    t, _ = jax.lax.fori_loop(0, 14, body_t,
                             (jnp.zeros((B, 1), jnp.int32),
                              jnp.full((B, 1), jnp.int32(N))))

    idx = jax.lax.broadcasted_iota(jnp.int32, (B, N), 1)
    sel_neg = ((ikey > v) | (eq & (idx < t))) & (ikey >= 0)
    cls_neg = jnp.sum(jnp.where(sel_neg, key, 0.0))

    npf = jnp.sum(npos_row).astype(jnp.float32)
    boxes_loss = bsum_ref[...] / npf
    cls_loss = (cpos_ref[...] + cls_neg) / npf
    bl_ref[...] = boxes_loss
    cl_ref[...] = cls_loss
    tot_ref[...] = boxes_loss + cls_loss


def kernel(pred_boxes, pred_logits, gt_boxes, gt_labels):
    labels = gt_labels.astype(jnp.int32)
    lgT = jnp.transpose(pred_logits, (2, 0, 1))
    pbT = jnp.transpose(pred_boxes, (0, 2, 1))
    gbT = jnp.transpose(gt_boxes, (0, 2, 1))

    key, npos, bsum, cpos = pl.pallas_call(
        _pass1_kernel,
        grid=(B // BB, NCHUNK),
        in_specs=[
            pl.BlockSpec((C, BB, NB), lambda i, j: (0, i, j)),
            pl.BlockSpec((BB, NB), lambda i, j: (i, j)),
            pl.BlockSpec((BB, 4, NB), lambda i, j: (i, 0, j)),
            pl.BlockSpec((BB, 4, NB), lambda i, j: (i, 0, j)),
        ],
        out_specs=[
            pl.BlockSpec((BB, NB), lambda i, j: (i, j)),
            pl.BlockSpec((BB, 1), lambda i, j: (i, 0)),
            pl.BlockSpec((1, 1), lambda i, j: (0, 0)),
            pl.BlockSpec((1, 1), lambda i, j: (0, 0)),
        ],
        out_shape=[
            jax.ShapeDtypeStruct((B, N), jnp.float32),
            jax.ShapeDtypeStruct((B, 1), jnp.int32),
            jax.ShapeDtypeStruct((1, 1), jnp.float32),
            jax.ShapeDtypeStruct((1, 1), jnp.float32),
        ],
    )(lgT, labels, pbT, gbT)

    bl, cl, tot = pl.pallas_call(
        _pass2_kernel,
        out_shape=[
            jax.ShapeDtypeStruct((1, 1), jnp.float32),
            jax.ShapeDtypeStruct((1, 1), jnp.float32),
            jax.ShapeDtypeStruct((1, 1), jnp.float32),
        ],
    )(key, npos, bsum, cpos)

    return (bl[0, 0], cl[0, 0], tot[0, 0])
